# baseline (speedup 1.0000x reference)
import functools

import jax
import jax.numpy as jnp
from jax import lax
from jax.experimental import pallas as pl
from jax.experimental.pallas import tpu as pltpu
from jax.experimental.pallas import tpu_sc as plsc

_INFO = plsc.get_sparse_core_info()
_NC = _INFO.num_cores
_NS = _INFO.num_subcores
_NW = _NC * _NS
_LANES = _INFO.num_lanes


def _detile_tables(tables, V, D):
    SK = D + 1
    NB = V // 128
    REM = V - NB * 128
    G = 5
    NLO = NB // _NW
    NHI = NB - NLO * _NW
    NMAX = NLO + (1 if NHI else 0)
    ROUNDS = (NMAX + G - 1) // G
    TW = 128 * G
    BLK = 128 * D

    mesh = plsc.VectorSubcoreMesh(core_axis_name="c", subcore_axis_name="s")

    @functools.partial(
        pl.kernel,
        out_type=tuple(jax.ShapeDtypeStruct((V * D,), jnp.float32)
                       for _ in tables),
        mesh=mesh,
        compiler_params=pltpu.CompilerParams(
            needs_layout_passes=False, use_tc_tiling_on_sc=True),
        scratch_types=(
            [pltpu.VMEM((D, TW), jnp.float32) for _ in range(2)]
            + [pltpu.VMEM((D, REM), jnp.float32)]
            + [pltpu.VMEM((128 * SK,), jnp.float32) for _ in range(2)]
            + [pltpu.VMEM((G * BLK,), jnp.float32) for _ in range(2)]
            + [pltpu.SemaphoreType.DMA for _ in range(4)]
        ),
    )
    def _detile(*refs):
        xts = refs[:3]
        outs = refs[3:6]
        (t0, t1, ptile, sk0, sk1, l0, l1,
         semt0, semt1, semo0, semo1) = refs[6:]
        tiles = (t0, t1)
        skew = (sk0, sk1)
        lin = (l0, l1)
        semt = (semt0, semt1)
        semo = (semo0, semo1)

        wid = lax.axis_index("s") * _NC + lax.axis_index("c")
        lo = wid * NLO + jnp.minimum(wid, NHI)
        n = NLO + jnp.where(wid < NHI, 1, 0)

        iota_sk = lax.broadcasted_iota(jnp.int32, (_LANES,), 0) * SK

        def bs(r):
            return lo + jnp.minimum(G * r, n - G)

        def fire(xt, r, p):
            co = pl.multiple_of(bs(r) * 128, 128)
            pltpu.async_copy(
                xt.at[pl.ds(0, D), pl.ds(co, TW)], tiles[p], semt[p])

        def drain(xt, r, p):
            co = pl.multiple_of(bs(r) * 128, 128)
            pltpu.make_async_copy(
                xt.at[pl.ds(0, D), pl.ds(co, TW)], tiles[p], semt[p]).wait()

        def compute(xt, out, r, p):
            @pl.when(r >= 2)
            def _():
                oo = pl.multiple_of(bs(r - 2) * BLK, 8)
                pltpu.make_async_copy(
                    lin[p], out.at[pl.ds(oo, G * BLK)], semo[p]).wait()

            def subcol(j, _):
                def qloop(q, _):
                    idxb = iota_sk + q
                    for k in range(8):
                        v = tiles[p][q, pl.ds(j * 128 + k * _LANES, _LANES)]
                        plsc.store_scatter(
                            skew[p], [idxb + k * _LANES * SK], v)
                    return 0

                lax.fori_loop(0, D, qloop, 0)

                def rloop(jj, _):
                    for u in range(4):
                        row = jj * 4 + u
                        for h in range(D // _LANES):
                            lin[p][pl.ds(j * BLK + row * D + h * _LANES,
                                         _LANES)] = \
                                skew[p][pl.ds(row * SK + h * _LANES, _LANES)]
                    return 0

                lax.fori_loop(0, 32, rloop, 0)
                return 0

            lax.fori_loop(0, G, subcol, 0)
            oo = pl.multiple_of(bs(r) * BLK, 8)
            pltpu.async_copy(lin[p], out.at[pl.ds(oo, G * BLK)], semo[p])

        for t, (xt, out) in enumerate(zip(xts, outs)):
            fire(xt, 0, 0)
            for r in range(ROUNDS):
                if r + 1 < ROUNDS:
                    fire(xt, r + 1, (r + 1) % 2)
                drain(xt, r, r % 2)
                compute(xt, out, r, r % 2)
            for r in (ROUNDS - 2, ROUNDS - 1):
                pltpu.make_async_copy(
                    lin[r % 2],
                    out.at[pl.ds(pl.multiple_of(bs(r) * BLK, 8), G * BLK)],
                    semo[r % 2]).wait()

            @pl.when(wid == _NW - 3 + t)
            def _(xt=xt, out=out):
                pltpu.async_copy(
                    xt.at[pl.ds(0, D), pl.ds(NB * 128, REM)],
                    ptile, semt0).wait()

                def pq(q, _):
                    idxb = iota_sk + q
                    for k in range(REM // _LANES):
                        v = ptile[q, pl.ds(k * _LANES, _LANES)]
                        plsc.store_scatter(skew[0], [idxb + k * _LANES * SK], v)
                    return 0

                lax.fori_loop(0, D, pq, 0)

                def pr(jj, _):
                    for u in range(4):
                        row = jj * 4 + u
                        for h in range(D // _LANES):
                            lin[0][pl.ds(row * D + h * _LANES, _LANES)] = \
                                skew[0][pl.ds(row * SK + h * _LANES, _LANES)]
                    return 0

                lax.fori_loop(0, REM // 4, pr, 0)
                pltpu.sync_copy(lin[0].at[pl.ds(0, REM * D)],
                                out.at[pl.ds(NB * 128 * D, REM * D)])

    return _detile(*[t.T for t in tables])


def kernel(user_id, item_id, hist_item, price, user_table, item_table,
           hist_table):
    B = user_id.shape[0]
    L = hist_item.shape[1]
    V = user_table.shape[0]
    D = user_table.shape[1]
    OUTW = 3 * D + 1
    RPW = B // _NW
    CB = 16
    NCH = RPW // CB
    NH = D // _LANES
    HPC = CB * L
    GS = [(k * 128, min(128, HPC - k * 128)) for k in range((HPC + 127) // 128)]

    hist_flat = hist_item.reshape(-1)
    utl, itl, htl = _detile_tables((user_table, item_table, hist_table), V, D)
    ut2 = utl.reshape(V, D)
    it2 = itl.reshape(V, D)
    ht2 = htl.reshape(V, D)

    mesh = plsc.VectorSubcoreMesh(core_axis_name="c", subcore_axis_name="s")

    @functools.partial(
        pl.kernel,
        out_type=jax.ShapeDtypeStruct((B, OUTW), jnp.float32),
        mesh=mesh,
        compiler_params=pltpu.CompilerParams(
            needs_layout_passes=False, use_tc_tiling_on_sc=False),
        scratch_types=[
            pltpu.VMEM((RPW,), jnp.int32),
            pltpu.VMEM((RPW,), jnp.int32),
            pltpu.VMEM((RPW * L,), jnp.int32),
            pltpu.VMEM((RPW,), jnp.float32),
            pltpu.VMEM((CB, D), jnp.float32),
            pltpu.VMEM((CB, D), jnp.float32),
            pltpu.VMEM((CB, D), jnp.float32),
            pltpu.VMEM((CB, D), jnp.float32),
            pltpu.VMEM((HPC, D), jnp.float32),
            pltpu.VMEM((HPC, D), jnp.float32),
            pltpu.VMEM((CB, OUTW), jnp.float32),
            pltpu.VMEM((CB, OUTW), jnp.float32),
            pltpu.SemaphoreType.DMA,
            pltpu.SemaphoreType.DMA,
            pltpu.SemaphoreType.DMA,
            pltpu.SemaphoreType.DMA,
            pltpu.SemaphoreType.DMA,
            pltpu.SemaphoreType.DMA,
        ],
    )
    def _emb(uid, iid, hid, pr, ut, it, ht, out,
             uidx, iidx, hidx, pst, su0, su1, si0, si1, hb0, hb1,
             st0, st1, semh0, semh1, semg0, semg1, semo0, semo1):
        su = (su0, su1)
        si = (si0, si1)
        hb = (hb0, hb1)
        st = (st0, st1)
        semh = (semh0, semh1)
        semg = (semg0, semg1)
        semo = (semo0, semo1)

        wid = lax.axis_index("s") * _NC + lax.axis_index("c")
        base = wid * RPW

        cps = [
            pltpu.async_copy(uid.at[pl.ds(base, RPW)], uidx, semg0),
            pltpu.async_copy(iid.at[pl.ds(base, RPW)], iidx, semg0),
            pltpu.async_copy(hid.at[pl.ds(base * L, RPW * L)], hidx, semg0),
            pltpu.async_copy(pr.at[pl.ds(base, RPW)], pst, semg0),
        ]
        for c in cps:
            c.wait()

        iota16 = lax.broadcasted_iota(jnp.int32, (_LANES,), 0)
        col_last = jnp.full((_LANES,), OUTW - 1, jnp.int32)

        def fire(g, p):
            r0 = g * CB
            h0 = r0 * L
            for (o, n) in GS:
                pltpu.async_copy(ht.at[hidx.at[pl.ds(h0 + o, n)]],
                                 hb[p].at[pl.ds(o, n)], semh[p])
            pltpu.async_copy(ut.at[uidx.at[pl.ds(r0, CB)]], su[p], semg[p])
            pltpu.async_copy(it.at[iidx.at[pl.ds(r0, CB)]], si[p], semg[p])

        def drain(g, p):
            r0 = g * CB
            h0 = r0 * L
            for (o, n) in GS:
                pltpu.make_async_copy(ht.at[hidx.at[pl.ds(h0 + o, n)]],
                                      hb[p].at[pl.ds(o, n)], semh[p]).wait()
            pltpu.make_async_copy(
                ut.at[uidx.at[pl.ds(r0, CB)]], su[p], semg[p]).wait()
            pltpu.make_async_copy(
                it.at[iidx.at[pl.ds(r0, CB)]], si[p], semg[p]).wait()

        def compute(g, p):
            r0 = g * CB

            @pl.when(g >= 2)
            def _():
                pltpu.make_async_copy(
                    st[p], out.at[pl.ds(base + (g - 2) * CB, CB)],
                    semo[p]).wait()

            def crow(c, _):
                for h in range(NH):
                    o = h * _LANES
                    acc = [hb[p][c * L + l, pl.ds(o, _LANES)]
                           for l in range(4)]
                    for l in range(4, L):
                        acc[l % 4] = (acc[l % 4]
                                      + hb[p][c * L + l, pl.ds(o, _LANES)])
                    a = (acc[0] + acc[1]) + (acc[2] + acc[3])
                    st[p][c, pl.ds(2 * D + o, _LANES)] = a
                    st[p][c, pl.ds(o, _LANES)] = su[p][c, pl.ds(o, _LANES)]
                    st[p][c, pl.ds(D + o, _LANES)] = si[p][c, pl.ds(o, _LANES)]
                return 0

            lax.fori_loop(0, CB, crow, 0)

            plsc.store_scatter(st[p], [iota16, col_last], pst[pl.ds(r0, CB)])
            pltpu.async_copy(st[p], out.at[pl.ds(base + r0, CB)], semo[p])

        NP = NCH // 2
        fire(0, 0)

        def pair(gp, _):
            g0 = gp * 2
            fire(g0 + 1, 1)
            drain(g0, 0)
            compute(g0, 0)

            @pl.when(gp < NP - 1)
            def _():
                fire(g0 + 2, 0)

            drain(g0 + 1, 1)
            compute(g0 + 1, 1)
            return 0

        lax.fori_loop(0, NP, pair, 0)

        pltpu.make_async_copy(
            st0, out.at[pl.ds(base + (NCH - 2) * CB, CB)], semo0).wait()
        pltpu.make_async_copy(
            st1, out.at[pl.ds(base + (NCH - 1) * CB, CB)], semo1).wait()

    return _emb(user_id, item_id, hist_flat, price, ut2, it2, ht2)


# speedup vs baseline: 1.0775x; 1.0775x over previous
import functools

import jax
import jax.numpy as jnp
from jax import lax
from jax.experimental import pallas as pl
from jax.experimental.pallas import tpu as pltpu
from jax.experimental.pallas import tpu_sc as plsc

_INFO = plsc.get_sparse_core_info()
_NC = _INFO.num_cores
_NS = _INFO.num_subcores
_NW = _NC * _NS
_LANES = _INFO.num_lanes


def kernel(user_id, item_id, hist_item, price, user_table, item_table,
           hist_table):
    B = user_id.shape[0]
    L = hist_item.shape[1]
    V = user_table.shape[0]
    D = user_table.shape[1]
    OUTW = 3 * D + 1
    RPW = B // _NW
    CB = 16
    NCH = RPW // CB
    NH = D // _LANES
    HPC = CB * L
    GS = [(k * 128, min(128, HPC - k * 128)) for k in range((HPC + 127) // 128)]

    hist_flat = hist_item.reshape(-1)

    mesh = plsc.VectorSubcoreMesh(core_axis_name="c", subcore_axis_name="s")

    @functools.partial(
        pl.kernel,
        out_type=jax.ShapeDtypeStruct((B, OUTW), jnp.float32),
        mesh=mesh,
        compiler_params=pltpu.CompilerParams(
            needs_layout_passes=False, use_tc_tiling_on_sc=False),
        scratch_types=[
            pltpu.VMEM((RPW,), jnp.int32),
            pltpu.VMEM((RPW,), jnp.int32),
            pltpu.VMEM((RPW * L,), jnp.int32),
            pltpu.VMEM((RPW,), jnp.float32),
            pltpu.VMEM((CB, D), jnp.float32),
            pltpu.VMEM((CB, D), jnp.float32),
            pltpu.VMEM((CB, D), jnp.float32),
            pltpu.VMEM((CB, D), jnp.float32),
            pltpu.VMEM((HPC, D), jnp.float32),
            pltpu.VMEM((HPC, D), jnp.float32),
            pltpu.VMEM((CB, OUTW), jnp.float32),
            pltpu.VMEM((CB, OUTW), jnp.float32),
            pltpu.SemaphoreType.DMA,
            pltpu.SemaphoreType.DMA,
            pltpu.SemaphoreType.DMA,
            pltpu.SemaphoreType.DMA,
            pltpu.SemaphoreType.DMA,
            pltpu.SemaphoreType.DMA,
        ],
    )
    def _emb(uid, iid, hid, pr, ut, it, ht, out,
             uidx, iidx, hidx, pst, su0, su1, si0, si1, hb0, hb1,
             st0, st1, semh0, semh1, semg0, semg1, semo0, semo1):
        su = (su0, su1)
        si = (si0, si1)
        hb = (hb0, hb1)
        st = (st0, st1)
        semh = (semh0, semh1)
        semg = (semg0, semg1)
        semo = (semo0, semo1)

        wid = lax.axis_index("s") * _NC + lax.axis_index("c")
        base = wid * RPW

        cps = [
            pltpu.async_copy(uid.at[pl.ds(base, RPW)], uidx, semg0),
            pltpu.async_copy(iid.at[pl.ds(base, RPW)], iidx, semg0),
            pltpu.async_copy(hid.at[pl.ds(base * L, RPW * L)], hidx, semg0),
            pltpu.async_copy(pr.at[pl.ds(base, RPW)], pst, semg0),
        ]
        for c in cps:
            c.wait()

        iota16 = lax.broadcasted_iota(jnp.int32, (_LANES,), 0)
        col_last = jnp.full((_LANES,), OUTW - 1, jnp.int32)

        def fire(g, p):
            r0 = g * CB
            h0 = r0 * L
            for (o, n) in GS:
                pltpu.async_copy(ht.at[hidx.at[pl.ds(h0 + o, n)]],
                                 hb[p].at[pl.ds(o, n)], semh[p])
            pltpu.async_copy(ut.at[uidx.at[pl.ds(r0, CB)]], su[p], semg[p])
            pltpu.async_copy(it.at[iidx.at[pl.ds(r0, CB)]], si[p], semg[p])

        def drain(g, p):
            r0 = g * CB
            h0 = r0 * L
            for (o, n) in GS:
                pltpu.make_async_copy(ht.at[hidx.at[pl.ds(h0 + o, n)]],
                                      hb[p].at[pl.ds(o, n)], semh[p]).wait()
            pltpu.make_async_copy(
                ut.at[uidx.at[pl.ds(r0, CB)]], su[p], semg[p]).wait()
            pltpu.make_async_copy(
                it.at[iidx.at[pl.ds(r0, CB)]], si[p], semg[p]).wait()

        def compute(g, p):
            r0 = g * CB

            @pl.when(g >= 2)
            def _():
                pltpu.make_async_copy(
                    st[p], out.at[pl.ds(base + (g - 2) * CB, CB)],
                    semo[p]).wait()

            def crow(c, _):
                for h in range(NH):
                    o = h * _LANES
                    acc = [hb[p][c * L + l, pl.ds(o, _LANES)]
                           for l in range(4)]
                    for l in range(4, L):
                        acc[l % 4] = (acc[l % 4]
                                      + hb[p][c * L + l, pl.ds(o, _LANES)])
                    a = (acc[0] + acc[1]) + (acc[2] + acc[3])
                    st[p][c, pl.ds(2 * D + o, _LANES)] = a
                    st[p][c, pl.ds(o, _LANES)] = su[p][c, pl.ds(o, _LANES)]
                    st[p][c, pl.ds(D + o, _LANES)] = si[p][c, pl.ds(o, _LANES)]
                return 0

            lax.fori_loop(0, CB, crow, 0)

            plsc.store_scatter(st[p], [iota16, col_last], pst[pl.ds(r0, CB)])
            pltpu.async_copy(st[p], out.at[pl.ds(base + r0, CB)], semo[p])

        NP = NCH // 2
        fire(0, 0)

        def pair(gp, _):
            g0 = gp * 2
            fire(g0 + 1, 1)
            drain(g0, 0)
            compute(g0, 0)

            @pl.when(gp < NP - 1)
            def _():
                fire(g0 + 2, 0)

            drain(g0 + 1, 1)
            compute(g0 + 1, 1)
            return 0

        lax.fori_loop(0, NP, pair, 0)

        pltpu.make_async_copy(
            st0, out.at[pl.ds(base + (NCH - 2) * CB, CB)], semo0).wait()
        pltpu.make_async_copy(
            st1, out.at[pl.ds(base + (NCH - 1) * CB, CB)], semo1).wait()

    return _emb(user_id, item_id, hist_flat, price, user_table, item_table,
                hist_table)


# speedup vs baseline: 1.1964x; 1.1104x over previous
import functools

import jax
import jax.numpy as jnp
from jax import lax
from jax.experimental import pallas as pl
from jax.experimental.pallas import tpu as pltpu
from jax.experimental.pallas import tpu_sc as plsc

_INFO = plsc.get_sparse_core_info()
_NC = _INFO.num_cores
_NS = _INFO.num_subcores
_NW = _NC * _NS
_LANES = _INFO.num_lanes

_CP = pltpu.CompilerParams(needs_layout_passes=False,
                           use_tc_tiling_on_sc=False)
_MESH = plsc.VectorSubcoreMesh(core_axis_name="c", subcore_axis_name="s")


def _hist_sums(hist_flat, hist_table, B, L, V, D):
    RPW = B // _NW
    CB = 16
    NCH = RPW // CB
    NH = D // _LANES
    HPC = CB * L
    GS = [(k * 128, min(128, HPC - k * 128))
          for k in range((HPC + 127) // 128)]

    @functools.partial(
        pl.kernel,
        out_type=jax.ShapeDtypeStruct((B * D,), jnp.float32),
        mesh=_MESH,
        compiler_params=_CP,
        scratch_types=[
            pltpu.VMEM((RPW * L,), jnp.int32),
            pltpu.VMEM((HPC, D), jnp.float32),
            pltpu.VMEM((HPC, D), jnp.float32),
            pltpu.VMEM((CB * D,), jnp.float32),
            pltpu.VMEM((CB * D,), jnp.float32),
            pltpu.SemaphoreType.DMA,
            pltpu.SemaphoreType.DMA,
            pltpu.SemaphoreType.DMA,
            pltpu.SemaphoreType.DMA,
        ],
    )
    def _h(hid, ht, out, hidx, hb0, hb1, st0, st1,
           semh0, semh1, semo0, semo1):
        hb = (hb0, hb1)
        st = (st0, st1)
        semh = (semh0, semh1)
        semo = (semo0, semo1)

        wid = lax.axis_index("s") * _NC + lax.axis_index("c")
        base = wid * RPW
        pltpu.sync_copy(hid.at[pl.ds(base * L, RPW * L)], hidx)

        def fire(g, p):
            h0 = g * CB * L
            for (o, n) in GS:
                pltpu.async_copy(ht.at[hidx.at[pl.ds(h0 + o, n)]],
                                 hb[p].at[pl.ds(o, n)], semh[p])

        def drain(g, p):
            h0 = g * CB * L
            for (o, n) in GS:
                pltpu.make_async_copy(ht.at[hidx.at[pl.ds(h0 + o, n)]],
                                      hb[p].at[pl.ds(o, n)], semh[p]).wait()

        def compute(g, p):
            r0 = g * CB

            @pl.when(g >= 2)
            def _():
                pltpu.make_async_copy(
                    st[p], out.at[pl.ds((base + (g - 2) * CB) * D, CB * D)],
                    semo[p]).wait()

            def crow(c, _):
                for h in range(NH):
                    o = h * _LANES
                    acc = [hb[p][c * L + l, pl.ds(o, _LANES)]
                           for l in range(4)]
                    for l in range(4, L):
                        acc[l % 4] = (acc[l % 4]
                                      + hb[p][c * L + l, pl.ds(o, _LANES)])
                    st[p][pl.ds(c * D + o, _LANES)] = \
                        (acc[0] + acc[1]) + (acc[2] + acc[3])
                return 0

            lax.fori_loop(0, CB, crow, 0)
            pltpu.async_copy(
                st[p], out.at[pl.ds((base + r0) * D, CB * D)], semo[p])

        NP = NCH // 2
        fire(0, 0)

        def pair(gp, _):
            g0 = gp * 2
            fire(g0 + 1, 1)
            drain(g0, 0)
            compute(g0, 0)

            @pl.when(gp < NP - 1)
            def _():
                fire(g0 + 2, 0)

            drain(g0 + 1, 1)
            compute(g0 + 1, 1)
            return 0

        lax.fori_loop(0, NP, pair, 0)
        pltpu.make_async_copy(
            st0, out.at[pl.ds((base + (NCH - 2) * CB) * D, CB * D)],
            semo0).wait()
        pltpu.make_async_copy(
            st1, out.at[pl.ds((base + (NCH - 1) * CB) * D, CB * D)],
            semo1).wait()

    return _h(hist_flat, hist_table)


def kernel(user_id, item_id, hist_item, price, user_table, item_table,
           hist_table):
    B = user_id.shape[0]
    L = hist_item.shape[1]
    V = user_table.shape[0]
    D = user_table.shape[1]
    OUTW = 3 * D + 1
    RPW = B // _NW
    CB = 16
    NCH = RPW // CB
    NH = D // _LANES

    hist_flat = hist_item.reshape(-1)
    hsum = _hist_sums(hist_flat, hist_table, B, L, V, D)
    h2d = hsum.reshape(B, D)

    @functools.partial(
        pl.kernel,
        out_type=jax.ShapeDtypeStruct((B, OUTW), jnp.float32),
        mesh=_MESH,
        compiler_params=_CP,
        scratch_types=[
            pltpu.VMEM((RPW,), jnp.int32),
            pltpu.VMEM((RPW,), jnp.int32),
            pltpu.VMEM((RPW,), jnp.float32),
            pltpu.VMEM((CB, D), jnp.float32),
            pltpu.VMEM((CB, D), jnp.float32),
            pltpu.VMEM((CB, D), jnp.float32),
            pltpu.VMEM((CB, D), jnp.float32),
            pltpu.VMEM((CB, D), jnp.float32),
            pltpu.VMEM((CB, D), jnp.float32),
            pltpu.VMEM((CB, OUTW), jnp.float32),
            pltpu.VMEM((CB, OUTW), jnp.float32),
            pltpu.SemaphoreType.DMA,
            pltpu.SemaphoreType.DMA,
            pltpu.SemaphoreType.DMA,
            pltpu.SemaphoreType.DMA,
        ],
    )
    def _cat(uid, iid, pr, hs, ut, it, out,
             uidx, iidx, pst, su0, su1, si0, si1, sh0, sh1,
             st0, st1, semg0, semg1, semo0, semo1):
        su = (su0, su1)
        si = (si0, si1)
        sh = (sh0, sh1)
        st = (st0, st1)
        semg = (semg0, semg1)
        semo = (semo0, semo1)

        wid = lax.axis_index("s") * _NC + lax.axis_index("c")
        base = wid * RPW

        cps = [
            pltpu.async_copy(uid.at[pl.ds(base, RPW)], uidx, semg0),
            pltpu.async_copy(iid.at[pl.ds(base, RPW)], iidx, semg0),
            pltpu.async_copy(pr.at[pl.ds(base, RPW)], pst, semg0),
        ]
        for c in cps:
            c.wait()

        iota16 = lax.broadcasted_iota(jnp.int32, (_LANES,), 0)
        col_last = jnp.full((_LANES,), OUTW - 1, jnp.int32)

        def fire(g, p):
            r0 = g * CB
            pltpu.async_copy(ut.at[uidx.at[pl.ds(r0, CB)]], su[p], semg[p])
            pltpu.async_copy(it.at[iidx.at[pl.ds(r0, CB)]], si[p], semg[p])
            pltpu.async_copy(hs.at[pl.ds(base + r0, CB)], sh[p], semg[p])

        def drain(g, p):
            r0 = g * CB
            pltpu.make_async_copy(
                ut.at[uidx.at[pl.ds(r0, CB)]], su[p], semg[p]).wait()
            pltpu.make_async_copy(
                it.at[iidx.at[pl.ds(r0, CB)]], si[p], semg[p]).wait()
            pltpu.make_async_copy(
                hs.at[pl.ds(base + r0, CB)], sh[p], semg[p]).wait()

        def compute(g, p):
            r0 = g * CB

            @pl.when(g >= 2)
            def _():
                pltpu.make_async_copy(
                    st[p], out.at[pl.ds(base + (g - 2) * CB, CB)],
                    semo[p]).wait()

            def crow(c, _):
                for h in range(NH):
                    o = h * _LANES
                    st[p][c, pl.ds(o, _LANES)] = su[p][c, pl.ds(o, _LANES)]
                    st[p][c, pl.ds(D + o, _LANES)] = si[p][c, pl.ds(o, _LANES)]
                    st[p][c, pl.ds(2 * D + o, _LANES)] = \
                        sh[p][c, pl.ds(o, _LANES)]
                return 0

            lax.fori_loop(0, CB, crow, 0)
            plsc.store_scatter(st[p], [iota16, col_last], pst[pl.ds(r0, CB)])
            pltpu.async_copy(st[p], out.at[pl.ds(base + r0, CB)], semo[p])

        NP = NCH // 2
        fire(0, 0)

        def pair(gp, _):
            g0 = gp * 2
            fire(g0 + 1, 1)
            drain(g0, 0)
            compute(g0, 0)

            @pl.when(gp < NP - 1)
            def _():
                fire(g0 + 2, 0)

            drain(g0 + 1, 1)
            compute(g0 + 1, 1)
            return 0

        lax.fori_loop(0, NP, pair, 0)
        pltpu.make_async_copy(
            st0, out.at[pl.ds(base + (NCH - 2) * CB, CB)], semo0).wait()
        pltpu.make_async_copy(
            st1, out.at[pl.ds(base + (NCH - 1) * CB, CB)], semo1).wait()

    return _cat(user_id, item_id, price, h2d, user_table, item_table)
